# baseline (speedup 1.0000x reference)
import jax
import jax.numpy as jnp
from jax import lax
from jax.experimental import pallas as pl
from jax.experimental.pallas import tpu as pltpu

ROWS, COLS = 64, 32768
RB = 16
GRID = ROWS // RB
LANE = 128
STEPS = COLS // LANE
NACC = 8


def _body(x_ref, idx_ref, val_ref):
    m = [jnp.full((RB, LANE), -jnp.inf, jnp.float32) for _ in range(NACC)]
    mi = [jnp.zeros((RB, LANE), jnp.int32) for _ in range(NACC)]
    for k in range(STEPS):
        a = k % NACC
        v = x_ref[:, k * LANE:(k + 1) * LANE]
        upd = v > m[a]
        m[a] = jnp.where(upd, v, m[a])
        mi[a] = jnp.where(upd, k * LANE, mi[a])
    bm, bmi = m[0], mi[0]
    for a in range(1, NACC):
        take = (m[a] > bm) | ((m[a] == bm) & (mi[a] < bmi))
        bm = jnp.where(take, m[a], bm)
        bmi = jnp.where(take, mi[a], bmi)
    col = bmi + lax.broadcasted_iota(jnp.int32, (RB, LANE), 1)
    best = jnp.max(bm, axis=1, keepdims=True)
    cand = jnp.where(bm == best, col, jnp.int32(COLS))
    idx_ref[...] = jnp.min(cand, axis=1, keepdims=True)
    val_ref[...] = best


def kernel(scores):
    scores = pltpu.with_memory_space_constraint(scores, pltpu.MemorySpace.HBM)
    idx, val = pl.pallas_call(
        _body,
        grid=(GRID,),
        in_specs=[pl.BlockSpec((RB, COLS), lambda j: (j, 0))],
        out_specs=[
            pl.BlockSpec((RB, 1), lambda j: (j, 0)),
            pl.BlockSpec((RB, 1), lambda j: (j, 0)),
        ],
        out_shape=[
            jax.ShapeDtypeStruct((ROWS, 1), jnp.int32),
            jax.ShapeDtypeStruct((ROWS, 1), jnp.float32),
        ],
    )(scores)
    return (idx, val)


# speedup vs baseline: 1.1103x; 1.1103x over previous
import jax
import jax.numpy as jnp
from jax import lax
from jax.experimental import pallas as pl
from jax.experimental.pallas import tpu as pltpu

ROWS, COLS = 64, 32768
NSTREAM = 8
BK = 512
GRID = COLS // (BK * NSTREAM)
LANE = 128
STEPS = BK // LANE


def _body(*refs):
    x_refs = refs[:NSTREAM]
    idx_ref, val_ref, rm, rmi = refs[NSTREAM:]
    j = pl.program_id(0)

    @pl.when(j == 0)
    def _init():
        rm[...] = jnp.full((ROWS, LANE), -jnp.inf, jnp.float32)
        rmi[...] = jnp.zeros((ROWS, LANE), jnp.int32)

    m = rm[...]
    mi = rmi[...]
    for q in range(NSTREAM):
        base = (j * NSTREAM + q) * BK
        for k in range(STEPS):
            v = x_refs[q][:, k * LANE:(k + 1) * LANE]
            upd = v > m
            m = jnp.where(upd, v, m)
            mi = jnp.where(upd, base + k * LANE, mi)
    rm[...] = m
    rmi[...] = mi

    @pl.when(j == GRID - 1)
    def _finalize():
        mv = lax.transpose(rm[...], (1, 0))
        col = lax.transpose(rmi[...], (1, 0)) + lax.broadcasted_iota(
            jnp.int32, (LANE, ROWS), 0
        )
        best = jnp.max(mv, axis=0, keepdims=True)
        cand = jnp.where(mv == best, col, jnp.int32(COLS))
        idx_ref[...] = jnp.min(cand, axis=0, keepdims=True)
        val_ref[...] = best


def kernel(scores):
    scores = pltpu.with_memory_space_constraint(scores, pltpu.MemorySpace.HBM)
    in_spec = lambda q: pl.BlockSpec(
        (ROWS, BK), lambda j, q=q: (0, j * NSTREAM + q)
    )
    idx, val = pl.pallas_call(
        _body,
        grid=(GRID,),
        in_specs=[in_spec(q) for q in range(NSTREAM)],
        out_specs=[
            pl.BlockSpec((1, ROWS), lambda j: (0, 0)),
            pl.BlockSpec((1, ROWS), lambda j: (0, 0)),
        ],
        out_shape=[
            jax.ShapeDtypeStruct((1, ROWS), jnp.int32),
            jax.ShapeDtypeStruct((1, ROWS), jnp.float32),
        ],
        scratch_shapes=[
            pltpu.VMEM((ROWS, LANE), jnp.float32),
            pltpu.VMEM((ROWS, LANE), jnp.int32),
        ],
    )(*([scores] * NSTREAM))
    return (idx.reshape(ROWS, 1), val.reshape(ROWS, 1))


# speedup vs baseline: 1.4292x; 1.2872x over previous
import jax
import jax.numpy as jnp
from jax import lax
from jax.experimental import pallas as pl
from jax.experimental.pallas import tpu as pltpu

ROWS, COLS = 64, 32768
NSTREAM = 8
BK = 1024
GRID = COLS // (BK * NSTREAM)
LANE = 128
STEPS = BK // LANE


def _body(*refs):
    x_refs = refs[:NSTREAM]
    idx_ref, val_ref, rm, rmi = refs[NSTREAM:]
    j = pl.program_id(0)

    @pl.when(j == 0)
    def _init():
        rm[...] = jnp.full((ROWS, LANE), -jnp.inf, jnp.float32)
        rmi[...] = jnp.zeros((ROWS, LANE), jnp.int32)

    m = rm[...]
    mi = rmi[...]
    for q in range(NSTREAM):
        base = (j * NSTREAM + q) * BK
        for k in range(STEPS):
            v = x_refs[q][:, k * LANE:(k + 1) * LANE]
            upd = v > m
            m = jnp.where(upd, v, m)
            mi = jnp.where(upd, base + k * LANE, mi)
    rm[...] = m
    rmi[...] = mi

    @pl.when(j == GRID - 1)
    def _finalize():
        mv = lax.transpose(rm[...], (1, 0))
        col = lax.transpose(rmi[...], (1, 0)) + lax.broadcasted_iota(
            jnp.int32, (LANE, ROWS), 0
        )
        best = jnp.max(mv, axis=0, keepdims=True)
        cand = jnp.where(mv == best, col, jnp.int32(COLS))
        idx_ref[...] = jnp.min(cand, axis=0, keepdims=True)
        val_ref[...] = best


def kernel(scores):
    scores = pltpu.with_memory_space_constraint(scores, pltpu.MemorySpace.HBM)
    in_spec = lambda q: pl.BlockSpec(
        (ROWS, BK), lambda j, q=q: (0, j * NSTREAM + q)
    )
    idx, val = pl.pallas_call(
        _body,
        grid=(GRID,),
        in_specs=[in_spec(q) for q in range(NSTREAM)],
        out_specs=[
            pl.BlockSpec((1, ROWS), lambda j: (0, 0)),
            pl.BlockSpec((1, ROWS), lambda j: (0, 0)),
        ],
        out_shape=[
            jax.ShapeDtypeStruct((1, ROWS), jnp.int32),
            jax.ShapeDtypeStruct((1, ROWS), jnp.float32),
        ],
        scratch_shapes=[
            pltpu.VMEM((ROWS, LANE), jnp.float32),
            pltpu.VMEM((ROWS, LANE), jnp.int32),
        ],
    )(*([scores] * NSTREAM))
    return (idx.reshape(ROWS, 1), val.reshape(ROWS, 1))
